# Initial kernel scaffold; baseline (speedup 1.0000x reference)
import jax
import jax.numpy as jnp
from jax.experimental import pallas as pl


def kernel(edge_index, h, embed, W1, W2, g_mlp, b_mlp, g_out, b_out, Wp, bp):
    raise NotImplementedError("write your pallas kernel here")


# speedup vs baseline: 4.3149x; 4.3149x over previous
import functools

import jax
import jax.numpy as jnp
from jax import lax
from jax.experimental import pallas as pl
from jax.experimental.pallas import tpu as pltpu
import jax.experimental.pallas.tpu_sc as plsc

N = 10000
E2 = 320000
H = 256
HH = 128
V = 2000
O = 128
L = 5

NC, NS = 2, 16
NP = 10240
NODE_T = NP // NS
ECHUNK = 128
ET_CH = 160
EGRP = 32
ET = ET_CH * ECHUNK
EP = ET * NS

BLK = 1024
NB = NP // BLK

@functools.lru_cache(maxsize=None)
def _mesh():
    return plsc.VectorSubcoreMesh(core_axis_name="c", subcore_axis_name="s",
                                  num_cores=NC, num_subcores=NS)


def _embed_body(ecat_hbm, h2_hbm, out_hbm, idx_v, buf_v, sem):
    c = lax.axis_index("c")
    s = lax.axis_index("s")
    pltpu.sync_copy(h2_hbm.at[c, s], idx_v)
    base = c * NP + s * NODE_T
    for j in range(NODE_T // ECHUNK):
        pltpu.async_copy(ecat_hbm.at[idx_v.at[j]], buf_v, sem).wait()
        pltpu.sync_copy(buf_v, out_hbm.at[pl.ds(base + j * ECHUNK, ECHUNK)])


@functools.lru_cache(maxsize=None)
def _embed_call():
    return pl.kernel(
        _embed_body, mesh=_mesh(),
        out_type=jax.ShapeDtypeStruct((2 * NP, HH), jnp.float32),
        scratch_types=[
            pltpu.VMEM((NODE_T // ECHUNK, ECHUNK), jnp.int32),
            pltpu.VMEM((ECHUNK, HH), jnp.float32),
            pltpu.SemaphoreType.DMA,
        ])


def _sc_embed(ecat, h2):
    return _embed_call()(ecat, h2)


def _agg_body(xcat_hbm, src2_hbm, dst3_hbm, out_hbm,
              src_v, dst_v, buf_v, sem, acc_sh):
    c = lax.axis_index("c")
    s = lax.axis_index("s")
    def _zb(i, _):
        for k in range(HH // 16):
            buf_v[i, pl.ds(k * 16, 16)] = jnp.zeros((16,), jnp.float32)
        return 0
    lax.fori_loop(0, ECHUNK, _zb, 0)
    for k in range(NODE_T // ECHUNK):
        pltpu.sync_copy(buf_v, acc_sh.at[pl.ds(s * NODE_T + k * ECHUNK, ECHUNK)])
    plsc.subcore_barrier()

    def _edge(j, _):
        pltpu.async_copy(xcat_hbm.at[src_v.at[j]], buf_v, sem).wait()
        pltpu.sync_copy(buf_v, acc_sh.at[dst_v.at[j]], add=True)
        return 0

    for g in range(ET_CH // EGRP):
        pltpu.sync_copy(src2_hbm.at[c, s, pl.ds(g * EGRP, EGRP)], src_v)
        pltpu.sync_copy(dst3_hbm.at[s, pl.ds(g * EGRP, EGRP)], dst_v)
        lax.fori_loop(0, EGRP, _edge, 0)
    plsc.subcore_barrier()
    pltpu.sync_copy(acc_sh.at[pl.ds(s * NODE_T, NODE_T)],
                    out_hbm.at[pl.ds(c * NP + s * NODE_T, NODE_T)])


@functools.lru_cache(maxsize=None)
def _agg_call():
    return pl.kernel(
        _agg_body, mesh=_mesh(),
        out_type=jax.ShapeDtypeStruct((2 * NP, HH), jnp.float32),
        scratch_types=[
            pltpu.VMEM((EGRP, ECHUNK), jnp.int32),
            pltpu.VMEM((EGRP, ECHUNK), jnp.int32),
            pltpu.VMEM((ECHUNK, HH), jnp.float32),
            pltpu.SemaphoreType.DMA,
            pltpu.VMEM_SHARED((NP, HH), jnp.float32),
        ])


def _sc_agg(xcat, src2, dst3):
    return _agg_call()(xcat, src2, dst3)


def _rows_mask(pid):
    rows = lax.broadcasted_iota(jnp.int32, (BLK, 1), 0) + pid * BLK
    return (rows < N).astype(jnp.float32)


def _a_body(xlo, xhi, alo, ahi, w_ref, z1_ref, st_ref, xs_ref):
    pid = pl.program_id(0)
    m = _rows_mask(pid)
    x = jnp.concatenate([xlo[...], xhi[...]], axis=1) * m
    agg = jnp.concatenate([alo[...], ahi[...]], axis=1) * m
    z0 = 2.0 * x + agg
    z1 = jnp.dot(z0, w_ref[...], preferred_element_type=jnp.float32)
    z1_ref[...] = z1
    st = jnp.concatenate([jnp.sum(z1, axis=0, keepdims=True),
                          jnp.sum(z1 * z1, axis=0, keepdims=True)], axis=0)
    xs = jnp.sum(x, axis=0, keepdims=True)

    @pl.when(pid == 0)
    def _():
        st_ref[...] = st
        xs_ref[...] = xs

    @pl.when(pid > 0)
    def _():
        st_ref[...] += st
        xs_ref[...] += xs


def _bn_scale(st, g, b):
    mean = st[0:1] / N
    var = st[1:2] / N - mean * mean
    s = g * lax.rsqrt(var + 1e-5)
    return s, b - mean * s


def _b_body(z1_ref, st1_ref, gb_ref, w_ref, z2_ref, st_ref):
    pid = pl.program_id(0)
    s, t = _bn_scale(st1_ref[...], gb_ref[0:1], gb_ref[1:2])
    zn = jnp.maximum(z1_ref[...] * s + t, 0.0) * _rows_mask(pid)
    z2 = jnp.dot(zn, w_ref[...], preferred_element_type=jnp.float32)
    z2_ref[...] = z2
    st = jnp.concatenate([jnp.sum(z2, axis=0, keepdims=True),
                          jnp.sum(z2 * z2, axis=0, keepdims=True)], axis=0)

    @pl.when(pid == 0)
    def _():
        st_ref[...] = st

    @pl.when(pid > 0)
    def _():
        st_ref[...] += st


def _c_body(z2_ref, st2_ref, gb_ref, x_ref, p_ref):
    i = pl.program_id(1)
    s, t = _bn_scale(st2_ref[...], gb_ref[0:1], gb_ref[1:2])
    xn = jnp.maximum(z2_ref[...] * s + t, 0.0) * _rows_mask(i)
    x_ref[...] = xn
    p = jnp.sum(xn, axis=0, keepdims=True)

    @pl.when(i == 0)
    def _():
        p_ref[...] = p

    @pl.when(i > 0)
    def _():
        p_ref[...] += p


def _score_body(pf_ref, wf_ref, bp_ref, out_ref):
    out_ref[...] = (jnp.dot(pf_ref[...], wf_ref[...],
                            preferred_element_type=jnp.float32)
                    + jnp.sum(bp_ref[...], axis=0, keepdims=True))


def _tc_a(xcat, agg, w1):
    return pl.pallas_call(
        _a_body,
        grid=(NB,),
        in_specs=[
            pl.BlockSpec((BLK, HH), lambda i: (i, 0)),
            pl.BlockSpec((BLK, HH), lambda i: (i + NB, 0)),
            pl.BlockSpec((BLK, HH), lambda i: (i, 0)),
            pl.BlockSpec((BLK, HH), lambda i: (i + NB, 0)),
            pl.BlockSpec((H, H), lambda i: (0, 0)),
        ],
        out_specs=[
            pl.BlockSpec((BLK, H), lambda i: (i, 0)),
            pl.BlockSpec((2, H), lambda i: (0, 0)),
            pl.BlockSpec((1, H), lambda i: (0, 0)),
        ],
        out_shape=[
            jax.ShapeDtypeStruct((NP, H), jnp.float32),
            jax.ShapeDtypeStruct((2, H), jnp.float32),
            jax.ShapeDtypeStruct((1, H), jnp.float32),
        ],
    )(xcat, xcat, agg, agg, w1)


def _tc_b(z1, st1, gb, w2):
    return pl.pallas_call(
        _b_body,
        grid=(NB,),
        in_specs=[
            pl.BlockSpec((BLK, H), lambda i: (i, 0)),
            pl.BlockSpec((2, H), lambda i: (0, 0)),
            pl.BlockSpec((2, H), lambda i: (0, 0)),
            pl.BlockSpec((H, H), lambda i: (0, 0)),
        ],
        out_specs=[
            pl.BlockSpec((BLK, H), lambda i: (i, 0)),
            pl.BlockSpec((2, H), lambda i: (0, 0)),
        ],
        out_shape=[
            jax.ShapeDtypeStruct((NP, H), jnp.float32),
            jax.ShapeDtypeStruct((2, H), jnp.float32),
        ],
    )(z1, st1, gb, w2)


def _tc_c(z2, st2, gb):
    return pl.pallas_call(
        _c_body,
        grid=(2, NB),
        in_specs=[
            pl.BlockSpec((BLK, HH), lambda hh, i: (i, hh)),
            pl.BlockSpec((2, HH), lambda hh, i: (0, hh)),
            pl.BlockSpec((2, HH), lambda hh, i: (0, hh)),
        ],
        out_specs=[
            pl.BlockSpec((BLK, HH), lambda hh, i: (hh * NB + i, 0)),
            pl.BlockSpec((1, HH), lambda hh, i: (0, hh)),
        ],
        out_shape=[
            jax.ShapeDtypeStruct((2 * NP, HH), jnp.float32),
            jax.ShapeDtypeStruct((1, H), jnp.float32),
        ],
    )(z2, st2, gb)


def _tc_score(pooled_flat, wp_flat, bp):
    return pl.pallas_call(
        _score_body,
        out_shape=jax.ShapeDtypeStruct((1, O), jnp.float32),
    )(pooled_flat, wp_flat, bp)


def kernel(edge_index, h, embed, W1, W2, g_mlp, b_mlp, g_out, b_out, Wp, bp):
    u, v = edge_index[0], edge_index[1]
    src = jnp.concatenate([u, v])
    dst = jnp.concatenate([v, u])
    pad = EP - E2
    srcp = jnp.concatenate([src, jnp.zeros((pad,), jnp.int32)])
    dstp = jnp.concatenate([dst, jnp.full((pad,), N, jnp.int32)])
    src2 = jnp.stack([srcp, srcp + NP]).reshape(NC, NS, ET_CH, ECHUNK)
    dst3 = dstp.reshape(NS, ET_CH, ECHUNK)

    hp = jnp.concatenate([h, jnp.zeros((NP - N,), jnp.int32)])
    h2 = jnp.stack([hp, hp + V]).reshape(NC, NS, NODE_T // ECHUNK, ECHUNK)
    ecat = jnp.concatenate([embed[:, :HH], embed[:, HH:]], axis=0)

    xcat = _sc_embed(ecat, h2)

    pooled = []
    for i in range(L - 1):
        agg = _sc_agg(xcat, src2, dst3)
        gb1 = jnp.stack([g_mlp[i], b_mlp[i]])
        gb2 = jnp.stack([g_out[i], b_out[i]])
        z1, st1, xs = _tc_a(xcat, agg, W1[i])
        if i == 0:
            pooled.append(xs)
        z2, st2 = _tc_b(z1, st1, gb1, W2[i])
        xcat, p = _tc_c(z2, st2, gb2)
        pooled.append(p)

    pooled_flat = jnp.concatenate(pooled, axis=0).reshape(1, L * H)
    wp_flat = Wp.reshape(L * H, O)
    return _tc_score(pooled_flat, wp_flat, bp)


# speedup vs baseline: 5.2034x; 1.2059x over previous
import functools

import jax
import jax.numpy as jnp
from jax import lax
from jax.experimental import pallas as pl
from jax.experimental.pallas import tpu as pltpu
import jax.experimental.pallas.tpu_sc as plsc

N = 10000
E2 = 320000
H = 256
HH = 128
V = 2000
O = 128
L = 5

NC, NS = 2, 16
NP = 10240
NODE_T = NP // NS
ECHUNK = 128
ET_CH = 160
EGRP = 32
ET = ET_CH * ECHUNK
EP = ET * NS

BLK = 1024
NB = NP // BLK

@functools.lru_cache(maxsize=None)
def _mesh():
    return plsc.VectorSubcoreMesh(core_axis_name="c", subcore_axis_name="s",
                                  num_cores=NC, num_subcores=NS)


def _embed_body(ecat_hbm, h2_hbm, out_hbm, idx_v, buf_v, sem):
    c = lax.axis_index("c")
    s = lax.axis_index("s")
    pltpu.sync_copy(h2_hbm.at[c, s], idx_v)
    base = c * NP + s * NODE_T
    for j in range(NODE_T // ECHUNK):
        pltpu.async_copy(ecat_hbm.at[idx_v.at[j]], buf_v, sem).wait()
        pltpu.sync_copy(buf_v, out_hbm.at[pl.ds(base + j * ECHUNK, ECHUNK)])


@functools.lru_cache(maxsize=None)
def _embed_call():
    return pl.kernel(
        _embed_body, mesh=_mesh(),
        out_type=jax.ShapeDtypeStruct((2 * NP, HH), jnp.float32),
        scratch_types=[
            pltpu.VMEM((NODE_T // ECHUNK, ECHUNK), jnp.int32),
            pltpu.VMEM((ECHUNK, HH), jnp.float32),
            pltpu.SemaphoreType.DMA,
        ])


def _sc_embed(ecat, h2):
    return _embed_call()(ecat, h2)


def _agg_body(xcat_hbm, src2_hbm, dst3_hbm, out_hbm,
              src_v, dst_v, buf_a, buf_b, sem_a, sem_b, acc_sh):
    c = lax.axis_index("c")
    s = lax.axis_index("s")
    def _zb(i, _):
        for k in range(HH // 16):
            buf_a[i, pl.ds(k * 16, 16)] = jnp.zeros((16,), jnp.float32)
        return 0
    lax.fori_loop(0, ECHUNK, _zb, 0)
    for k in range(NODE_T // ECHUNK):
        pltpu.sync_copy(buf_a, acc_sh.at[pl.ds(s * NODE_T + k * ECHUNK, ECHUNK)])
    plsc.subcore_barrier()

    def _fire(j, buf, sem):
        pltpu.async_copy(xcat_hbm.at[src_v.at[j]], buf, sem)

    def _drain(j, buf, sem):
        pltpu.make_async_copy(xcat_hbm.at[src_v.at[j]], buf, sem).wait()
        pltpu.sync_copy(buf, acc_sh.at[dst_v.at[j]], add=True)

    for g in range(ET_CH // EGRP):
        pltpu.sync_copy(src2_hbm.at[c, s, pl.ds(g * EGRP, EGRP)], src_v)
        pltpu.sync_copy(dst3_hbm.at[s, pl.ds(g * EGRP, EGRP)], dst_v)
        _fire(0, buf_a, sem_a)

        def _pair(p, _):
            _fire(2 * p + 1, buf_b, sem_b)
            _drain(2 * p, buf_a, sem_a)
            _fire(2 * p + 2, buf_a, sem_a)
            _drain(2 * p + 1, buf_b, sem_b)
            return 0
        lax.fori_loop(0, EGRP // 2 - 1, _pair, 0)
        _fire(EGRP - 1, buf_b, sem_b)
        _drain(EGRP - 2, buf_a, sem_a)
        _drain(EGRP - 1, buf_b, sem_b)
    plsc.subcore_barrier()
    pltpu.sync_copy(acc_sh.at[pl.ds(s * NODE_T, NODE_T)],
                    out_hbm.at[pl.ds(c * NP + s * NODE_T, NODE_T)])


@functools.lru_cache(maxsize=None)
def _agg_call():
    return pl.kernel(
        _agg_body, mesh=_mesh(),
        out_type=jax.ShapeDtypeStruct((2 * NP, HH), jnp.float32),
        scratch_types=[
            pltpu.VMEM((EGRP, ECHUNK), jnp.int32),
            pltpu.VMEM((EGRP, ECHUNK), jnp.int32),
            pltpu.VMEM((ECHUNK, HH), jnp.float32),
            pltpu.VMEM((ECHUNK, HH), jnp.float32),
            pltpu.SemaphoreType.DMA,
            pltpu.SemaphoreType.DMA,
            pltpu.VMEM_SHARED((NP, HH), jnp.float32),
        ])


def _sc_agg(xcat, src2, dst3):
    return _agg_call()(xcat, src2, dst3)


def _rows_mask(pid):
    rows = lax.broadcasted_iota(jnp.int32, (BLK, 1), 0) + pid * BLK
    return (rows < N).astype(jnp.float32)


def _a_body(xlo, xhi, alo, ahi, w_ref, z1_ref, st_ref, xs_ref):
    pid = pl.program_id(0)
    m = _rows_mask(pid)
    x = jnp.concatenate([xlo[...], xhi[...]], axis=1) * m
    agg = jnp.concatenate([alo[...], ahi[...]], axis=1) * m
    z0 = 2.0 * x + agg
    z1 = jnp.dot(z0, w_ref[...], preferred_element_type=jnp.float32)
    z1_ref[...] = z1
    st = jnp.concatenate([jnp.sum(z1, axis=0, keepdims=True),
                          jnp.sum(z1 * z1, axis=0, keepdims=True)], axis=0)
    xs = jnp.sum(x, axis=0, keepdims=True)

    @pl.when(pid == 0)
    def _():
        st_ref[...] = st
        xs_ref[...] = xs

    @pl.when(pid > 0)
    def _():
        st_ref[...] += st
        xs_ref[...] += xs


def _bn_scale(st, g, b):
    mean = st[0:1] / N
    var = st[1:2] / N - mean * mean
    s = g * lax.rsqrt(var + 1e-5)
    return s, b - mean * s


def _b_body(z1_ref, st1_ref, gb_ref, w_ref, z2_ref, st_ref):
    pid = pl.program_id(0)
    s, t = _bn_scale(st1_ref[...], gb_ref[0:1], gb_ref[1:2])
    zn = jnp.maximum(z1_ref[...] * s + t, 0.0) * _rows_mask(pid)
    z2 = jnp.dot(zn, w_ref[...], preferred_element_type=jnp.float32)
    z2_ref[...] = z2
    st = jnp.concatenate([jnp.sum(z2, axis=0, keepdims=True),
                          jnp.sum(z2 * z2, axis=0, keepdims=True)], axis=0)

    @pl.when(pid == 0)
    def _():
        st_ref[...] = st

    @pl.when(pid > 0)
    def _():
        st_ref[...] += st


def _c_body(z2_ref, st2_ref, gb_ref, x_ref, p_ref):
    i = pl.program_id(1)
    s, t = _bn_scale(st2_ref[...], gb_ref[0:1], gb_ref[1:2])
    xn = jnp.maximum(z2_ref[...] * s + t, 0.0) * _rows_mask(i)
    x_ref[...] = xn
    p = jnp.sum(xn, axis=0, keepdims=True)

    @pl.when(i == 0)
    def _():
        p_ref[...] = p

    @pl.when(i > 0)
    def _():
        p_ref[...] += p


def _score_body(pf_ref, wf_ref, bp_ref, out_ref):
    out_ref[...] = (jnp.dot(pf_ref[...], wf_ref[...],
                            preferred_element_type=jnp.float32)
                    + jnp.sum(bp_ref[...], axis=0, keepdims=True))


def _tc_a(xcat, agg, w1):
    return pl.pallas_call(
        _a_body,
        grid=(NB,),
        in_specs=[
            pl.BlockSpec((BLK, HH), lambda i: (i, 0)),
            pl.BlockSpec((BLK, HH), lambda i: (i + NB, 0)),
            pl.BlockSpec((BLK, HH), lambda i: (i, 0)),
            pl.BlockSpec((BLK, HH), lambda i: (i + NB, 0)),
            pl.BlockSpec((H, H), lambda i: (0, 0)),
        ],
        out_specs=[
            pl.BlockSpec((BLK, H), lambda i: (i, 0)),
            pl.BlockSpec((2, H), lambda i: (0, 0)),
            pl.BlockSpec((1, H), lambda i: (0, 0)),
        ],
        out_shape=[
            jax.ShapeDtypeStruct((NP, H), jnp.float32),
            jax.ShapeDtypeStruct((2, H), jnp.float32),
            jax.ShapeDtypeStruct((1, H), jnp.float32),
        ],
    )(xcat, xcat, agg, agg, w1)


def _tc_b(z1, st1, gb, w2):
    return pl.pallas_call(
        _b_body,
        grid=(NB,),
        in_specs=[
            pl.BlockSpec((BLK, H), lambda i: (i, 0)),
            pl.BlockSpec((2, H), lambda i: (0, 0)),
            pl.BlockSpec((2, H), lambda i: (0, 0)),
            pl.BlockSpec((H, H), lambda i: (0, 0)),
        ],
        out_specs=[
            pl.BlockSpec((BLK, H), lambda i: (i, 0)),
            pl.BlockSpec((2, H), lambda i: (0, 0)),
        ],
        out_shape=[
            jax.ShapeDtypeStruct((NP, H), jnp.float32),
            jax.ShapeDtypeStruct((2, H), jnp.float32),
        ],
    )(z1, st1, gb, w2)


def _tc_c(z2, st2, gb):
    return pl.pallas_call(
        _c_body,
        grid=(2, NB),
        in_specs=[
            pl.BlockSpec((BLK, HH), lambda hh, i: (i, hh)),
            pl.BlockSpec((2, HH), lambda hh, i: (0, hh)),
            pl.BlockSpec((2, HH), lambda hh, i: (0, hh)),
        ],
        out_specs=[
            pl.BlockSpec((BLK, HH), lambda hh, i: (hh * NB + i, 0)),
            pl.BlockSpec((1, HH), lambda hh, i: (0, hh)),
        ],
        out_shape=[
            jax.ShapeDtypeStruct((2 * NP, HH), jnp.float32),
            jax.ShapeDtypeStruct((1, H), jnp.float32),
        ],
    )(z2, st2, gb)


def _tc_score(pooled_flat, wp_flat, bp):
    return pl.pallas_call(
        _score_body,
        out_shape=jax.ShapeDtypeStruct((1, O), jnp.float32),
    )(pooled_flat, wp_flat, bp)


def kernel(edge_index, h, embed, W1, W2, g_mlp, b_mlp, g_out, b_out, Wp, bp):
    u, v = edge_index[0], edge_index[1]
    src = jnp.concatenate([u, v])
    dst = jnp.concatenate([v, u])
    pad = EP - E2
    srcp = jnp.concatenate([src, jnp.zeros((pad,), jnp.int32)])
    dstp = jnp.concatenate([dst, jnp.full((pad,), N, jnp.int32)])
    src2 = jnp.stack([srcp, srcp + NP]).reshape(NC, NS, ET_CH, ECHUNK)
    dst3 = dstp.reshape(NS, ET_CH, ECHUNK)

    hp = jnp.concatenate([h, jnp.zeros((NP - N,), jnp.int32)])
    h2 = jnp.stack([hp, hp + V]).reshape(NC, NS, NODE_T // ECHUNK, ECHUNK)
    ecat = jnp.concatenate([embed[:, :HH], embed[:, HH:]], axis=0)

    xcat = _sc_embed(ecat, h2)

    pooled = []
    for i in range(L - 1):
        agg = _sc_agg(xcat, src2, dst3)
        gb1 = jnp.stack([g_mlp[i], b_mlp[i]])
        gb2 = jnp.stack([g_out[i], b_out[i]])
        z1, st1, xs = _tc_a(xcat, agg, W1[i])
        if i == 0:
            pooled.append(xs)
        z2, st2 = _tc_b(z1, st1, gb1, W2[i])
        xcat, p = _tc_c(z2, st2, gb2)
        pooled.append(p)

    pooled_flat = jnp.concatenate(pooled, axis=0).reshape(1, L * H)
    wp_flat = Wp.reshape(L * H, O)
    return _tc_score(pooled_flat, wp_flat, bp)
